# Initial kernel scaffold; baseline (speedup 1.0000x reference)
import jax
import jax.numpy as jnp
from jax.experimental import pallas as pl


def kernel(global_feature, msf, W0, b0, W1, b1):
    raise NotImplementedError("write your pallas kernel here")


# speedup vs baseline: 5.5506x; 5.5506x over previous
import functools

import jax
import jax.numpy as jnp
from jax import lax
from jax.experimental import pallas as pl
from jax.experimental.pallas import tpu as pltpu
from jax.experimental.pallas import tpu_sc as plsc

K = 16


def _tc_body(msfT_ref, msf_ref, gf_ref, w0gT_ref, w0mT_ref, w1T_ref, b0_ref,
             b1_ref, zT_ref, idx_ref):
    b = pl.program_id(0)
    P = msfT_ref[0]
    X = msf_ref[0]
    N = P.shape[0]

    G = lax.dot_general(P, X, (((1,), (0,)), ((), ())),
                        preferred_element_type=jnp.float32)
    sqc = jnp.sum(P * P, axis=1, keepdims=True)
    sqr = jnp.sum(X * X, axis=0, keepdims=True)
    d2 = sqc + sqr - 2.0 * G

    iota = lax.broadcasted_iota(jnp.int32, (N, N), 1)
    lane_k = lax.broadcasted_iota(jnp.int32, (N, K), 1)
    acc0 = jnp.zeros((N, K), dtype=jnp.int32)

    def step(t, carry):
        d2c, acc = carry
        m = jnp.min(d2c, axis=1, keepdims=True)
        am = jnp.min(jnp.where(d2c <= m, iota, N), axis=1,
                     keepdims=True)
        acc = jnp.where(lane_k == t, am, acc)
        d2c = jnp.where(iota == am, jnp.inf, d2c)
        return d2c, acc

    _, acc = lax.fori_loop(0, K, step, (d2, acc0))
    idx_ref[0] = acc + b * N

    g0 = lax.dot_general(gf_ref[0], w0gT_ref[...], (((1,), (0,)), ((), ())),
                         preferred_element_type=jnp.float32,
                         precision=lax.Precision.HIGHEST)
    h = lax.dot_general(P, w0mT_ref[...], (((1,), (0,)), ((), ())),
                        preferred_element_type=jnp.float32,
                        precision=lax.Precision.HIGHEST)
    h = jnp.maximum(h + g0 + b0_ref[...], 0.0)
    z = lax.dot_general(h, w1T_ref[...], (((1,), (0,)), ((), ())),
                        preferred_element_type=jnp.float32,
                        precision=lax.Precision.HIGHEST)
    zT_ref[0] = jnp.maximum(z + b1_ref[...], 0.0)


def _tc_call(msfT, msf, gf3, w0gT, w0mT, w1T, b0r, b1r):
    B, N, C = msfT.shape
    H = w0mT.shape[1]
    O = w1T.shape[1]
    Cg = gf3.shape[2]
    full = lambda shape: pl.BlockSpec(shape, lambda b: (0,) * len(shape))
    return pl.pallas_call(
        _tc_body,
        grid=(B,),
        in_specs=[
            pl.BlockSpec((1, N, C), lambda b: (b, 0, 0)),
            pl.BlockSpec((1, C, N), lambda b: (b, 0, 0)),
            pl.BlockSpec((1, 1, Cg), lambda b: (b, 0, 0)),
            full((Cg, H)),
            full((C, H)),
            full((H, O)),
            full((1, H)),
            full((1, O)),
        ],
        out_specs=[
            pl.BlockSpec((1, N, O), lambda b: (b, 0, 0)),
            pl.BlockSpec((1, N, K), lambda b: (b, 0, 0)),
        ],
        out_shape=[
            jax.ShapeDtypeStruct((B, N, O), jnp.float32),
            jax.ShapeDtypeStruct((B, N, K), jnp.int32),
        ],
    )(msfT, msf, gf3, w0gT, w0mT, w1T, b0r, b1r)


_NC, _NS, _L = 2, 16, 16
_NW = _NC * _NS


def _sc_gather_max(idx_flat, z_flat):
    PTS, D = z_flat.shape
    PPW = PTS // _NW
    CP = 8
    NCH = PPW // CP
    ROWS = CP * K
    mesh = plsc.VectorSubcoreMesh(core_axis_name="c", subcore_axis_name="s")

    @functools.partial(
        pl.kernel, mesh=mesh,
        out_type=jax.ShapeDtypeStruct((PTS, D), jnp.float32),
        scratch_types=[
            pltpu.VMEM((PPW * K,), jnp.int32),
            pltpu.VMEM((ROWS, D), jnp.float32),
            pltpu.VMEM((CP, D), jnp.float32),
            pltpu.SemaphoreType.DMA,
        ],
    )
    def body(idx_hbm, z_hbm, out_hbm, idx_v, rows_v, outc_v, sem):
        wid = lax.axis_index("s") * _NC + lax.axis_index("c")
        pltpu.sync_copy(idx_hbm.at[pl.ds(wid * PPW * K, PPW * K)], idx_v)

        def chunk(c, _):
            pltpu.async_copy(z_hbm.at[idx_v.at[pl.ds(c * ROWS, ROWS)]],
                             rows_v, sem).wait()

            def point(p, _):
                def col(q, _):
                    m = rows_v[p * K, pl.ds(q * _L, _L)]
                    for r in range(1, K):
                        m = jnp.maximum(m, rows_v[p * K + r, pl.ds(q * _L, _L)])
                    outc_v[p, pl.ds(q * _L, _L)] = m
                    return 0

                lax.fori_loop(0, D // _L, col, 0)
                return 0

            lax.fori_loop(0, CP, point, 0)
            pltpu.sync_copy(outc_v,
                            out_hbm.at[pl.ds(wid * PPW + c * CP, CP)])
            return 0

        lax.fori_loop(0, NCH, chunk, 0)

    return body(idx_flat, z_flat)


def kernel(global_feature, msf, W0, b0, W1, b1):
    B, C, N = msf.shape
    Cg = global_feature.shape[1]
    msfT = jnp.transpose(msf, (0, 2, 1))
    gf3 = global_feature[:, None, :]
    w0gT = jnp.transpose(W0[:, :Cg])
    w0mT = jnp.transpose(W0[:, Cg:])
    w1T = jnp.transpose(W1)
    zT, idx = _tc_call(msfT, msf, gf3, w0gT, w0mT, w1T, b0[None, :],
                       b1[None, :])
    out = _sc_gather_max(idx.reshape(-1), zT.reshape(B * N, -1))
    return jnp.transpose(out.reshape(B, N, -1), (0, 2, 1))


# speedup vs baseline: 5.6197x; 1.0124x over previous
import functools

import jax
import jax.numpy as jnp
from jax import lax
from jax.experimental import pallas as pl
from jax.experimental.pallas import tpu as pltpu
from jax.experimental.pallas import tpu_sc as plsc

K = 16


def _tc_body(msfT_ref, msf_ref, gf_ref, w0gT_ref, w0mT_ref, w1T_ref, b0_ref,
             b1_ref, zT_ref, idx_ref):
    b = pl.program_id(0)
    P = msfT_ref[0]
    X = msf_ref[0]
    N = P.shape[0]

    G = lax.dot_general(P, X, (((1,), (0,)), ((), ())),
                        preferred_element_type=jnp.float32)
    sqc = jnp.sum(P * P, axis=1, keepdims=True)
    sqr = jnp.sum(X * X, axis=0, keepdims=True)
    d2 = sqc + sqr - 2.0 * G

    acc0 = jnp.zeros((N, K), dtype=jnp.int32)

    def step(t, carry):
        d2c, acc = carry
        iota = lax.broadcasted_iota(jnp.int32, (N, N), 1)
        lane_k = lax.broadcasted_iota(jnp.int32, (N, K), 1)
        m = jnp.min(d2c, axis=1, keepdims=True)
        am = jnp.min(jnp.where(d2c <= m, iota, N), axis=1,
                     keepdims=True)
        acc = jnp.where(lane_k == t, am, acc)
        d2c = jnp.where(iota == am, jnp.inf, d2c)
        return d2c, acc

    _, acc = lax.fori_loop(0, K, step, (d2, acc0))
    idx_ref[0] = acc + b * N

    g0 = lax.dot_general(gf_ref[0], w0gT_ref[...], (((1,), (0,)), ((), ())),
                         preferred_element_type=jnp.float32,
                         precision=lax.Precision.HIGHEST)
    h = lax.dot_general(P, w0mT_ref[...], (((1,), (0,)), ((), ())),
                        preferred_element_type=jnp.float32,
                        precision=lax.Precision.HIGHEST)
    h = jnp.maximum(h + g0 + b0_ref[...], 0.0)
    z = lax.dot_general(h, w1T_ref[...], (((1,), (0,)), ((), ())),
                        preferred_element_type=jnp.float32,
                        precision=lax.Precision.HIGHEST)
    zT_ref[0] = jnp.maximum(z + b1_ref[...], 0.0)


def _tc_call(msfT, msf, gf3, w0gT, w0mT, w1T, b0r, b1r):
    B, N, C = msfT.shape
    H = w0mT.shape[1]
    O = w1T.shape[1]
    Cg = gf3.shape[2]
    full = lambda shape: pl.BlockSpec(shape, lambda b: (0,) * len(shape))
    return pl.pallas_call(
        _tc_body,
        grid=(B,),
        in_specs=[
            pl.BlockSpec((1, N, C), lambda b: (b, 0, 0)),
            pl.BlockSpec((1, C, N), lambda b: (b, 0, 0)),
            pl.BlockSpec((1, 1, Cg), lambda b: (b, 0, 0)),
            full((Cg, H)),
            full((C, H)),
            full((H, O)),
            full((1, H)),
            full((1, O)),
        ],
        out_specs=[
            pl.BlockSpec((1, N, O), lambda b: (b, 0, 0)),
            pl.BlockSpec((1, N, K), lambda b: (b, 0, 0)),
        ],
        out_shape=[
            jax.ShapeDtypeStruct((B, N, O), jnp.float32),
            jax.ShapeDtypeStruct((B, N, K), jnp.int32),
        ],
    )(msfT, msf, gf3, w0gT, w0mT, w1T, b0r, b1r)


_NC, _NS, _L = 2, 16, 16
_NW = _NC * _NS


def _sc_gather_max(idx_flat, z_flat):
    PTS, D = z_flat.shape
    PPW = PTS // _NW
    CP = 8
    NCH = PPW // CP
    ROWS = CP * K
    mesh = plsc.VectorSubcoreMesh(core_axis_name="c", subcore_axis_name="s")

    @functools.partial(
        pl.kernel, mesh=mesh,
        out_type=jax.ShapeDtypeStruct((PTS, D), jnp.float32),
        scratch_types=[
            pltpu.VMEM((PPW * K,), jnp.int32),
            pltpu.VMEM((ROWS, D), jnp.float32),
            pltpu.VMEM((ROWS, D), jnp.float32),
            pltpu.VMEM((CP, D), jnp.float32),
            pltpu.SemaphoreType.DMA,
            pltpu.SemaphoreType.DMA,
        ],
    )
    def body(idx_hbm, z_hbm, out_hbm, idx_v, rows0, rows1, outc_v, sem0,
             sem1):
        wid = lax.axis_index("s") * _NC + lax.axis_index("c")
        pltpu.sync_copy(idx_hbm.at[pl.ds(wid * PPW * K, PPW * K)], idx_v)
        bufs = (rows0, rows1)
        sems = (sem0, sem1)
        pltpu.async_copy(z_hbm.at[idx_v.at[pl.ds(0, ROWS)]], rows0, sem0)

        def pair(i, _):
            for par in range(2):
                c = i * 2 + par
                rows_v = bufs[par]
                @pl.when(c + 1 < NCH)
                def _():
                    pltpu.async_copy(
                        z_hbm.at[idx_v.at[pl.ds((c + 1) * ROWS, ROWS)]],
                        bufs[1 - par], sems[1 - par])

                pltpu.make_async_copy(
                    z_hbm.at[idx_v.at[pl.ds(c * ROWS, ROWS)]], rows_v,
                    sems[par]).wait()

                def point(p, _):
                    def col(q, _):
                        m = rows_v[p * K, pl.ds(q * _L, _L)]
                        for r in range(1, K):
                            m = jnp.maximum(
                                m, rows_v[p * K + r, pl.ds(q * _L, _L)])
                        outc_v[p, pl.ds(q * _L, _L)] = m
                        return 0

                    lax.fori_loop(0, D // _L, col, 0)
                    return 0

                lax.fori_loop(0, CP, point, 0)
                pltpu.sync_copy(outc_v,
                                out_hbm.at[pl.ds(wid * PPW + c * CP, CP)])
            return 0

        lax.fori_loop(0, NCH // 2, pair, 0)

    return body(idx_flat, z_flat)


def kernel(global_feature, msf, W0, b0, W1, b1):
    B, C, N = msf.shape
    Cg = global_feature.shape[1]
    msfT = jnp.transpose(msf, (0, 2, 1))
    gf3 = global_feature[:, None, :]
    w0gT = jnp.transpose(W0[:, :Cg])
    w0mT = jnp.transpose(W0[:, Cg:])
    w1T = jnp.transpose(W1)
    zT, idx = _tc_call(msfT, msf, gf3, w0gT, w0mT, w1T, b0[None, :],
                       b1[None, :])
    out = _sc_gather_max(idx.reshape(-1), zT.reshape(B * N, -1))
    return jnp.transpose(out.reshape(B, N, -1), (0, 2, 1))
